# Initial kernel scaffold; baseline (speedup 1.0000x reference)
import functools

import jax
import jax.numpy as jnp
from jax import lax
from jax.experimental import pallas as pl
from jax.experimental.pallas import tpu as pltpu
from jax.experimental.pallas import tpu_sc as plsc

HIDDEN = 1024
NTOK = 4 * 8192
NC, NS = 2, 16
NW = NC * NS
TPW = NTOK // NW
CH = 32
NCH = TPW // CH


def _emb_body(ids_hbm, pos_hbm, wtab_hbm, ptab_hbm, out_hbm,
              ids_v, pos_v, buf, sem_w, sem_p, sem_o):
    c = lax.axis_index("c")
    s = lax.axis_index("s")
    wid = c * NS + s
    base = wid * TPW

    pltpu.sync_copy(ids_hbm.at[pl.ds(base, TPW)], ids_v)
    pltpu.sync_copy(pos_hbm.at[pl.ds(base, TPW)], pos_v)

    w_desc = [None, None]
    s_desc = [None, None]

    def fire_w(k):
        b = k & 1
        w_desc[b] = pltpu.async_copy(
            wtab_hbm.at[ids_v.at[pl.ds(k * CH, CH)]], buf.at[b], sem_w)

    fire_w(0)
    for k in range(NCH):
        b = k & 1
        w_desc[b].wait()
        pltpu.async_copy(
            ptab_hbm.at[pos_v.at[pl.ds(k * CH, CH)]], buf.at[b], sem_p,
            add=True).wait()
        if s_desc[1 - b] is not None:
            s_desc[1 - b].wait()
        if k + 1 < NCH:
            fire_w(k + 1)
        s_desc[b] = pltpu.async_copy(
            buf.at[b], out_hbm.at[pl.ds(base + k * CH, CH)], sem_o)
    s_desc[(NCH - 1) & 1].wait()


@functools.partial(
    pl.kernel,
    out_type=jax.ShapeDtypeStruct((NTOK, HIDDEN), jnp.float32),
    mesh=plsc.VectorSubcoreMesh(core_axis_name="c", subcore_axis_name="s"),
    scratch_types=[
        pltpu.VMEM((TPW,), jnp.int32),
        pltpu.VMEM((TPW,), jnp.int32),
        pltpu.VMEM((2, CH, HIDDEN), jnp.float32),
        pltpu.SemaphoreType.DMA,
        pltpu.SemaphoreType.DMA,
        pltpu.SemaphoreType.DMA,
    ],
)
def _emb_call(ids_hbm, pos_hbm, wtab_hbm, ptab_hbm, out_hbm,
              ids_v, pos_v, buf, sem_w, sem_p, sem_o):
    _emb_body(ids_hbm, pos_hbm, wtab_hbm, ptab_hbm, out_hbm,
              ids_v, pos_v, buf, sem_w, sem_p, sem_o)


@jax.jit
def kernel(input_ids, position_ids, word_table, pos_table):
    bsh = input_ids.shape
    ids = input_ids.reshape(-1).astype(jnp.int32)
    pos = position_ids.reshape(-1).astype(jnp.int32)
    out = _emb_call(ids, pos, word_table, pos_table)
    return out.reshape(*bsh, HIDDEN)


# speedup vs baseline: 2.1476x; 2.1476x over previous
import functools

import jax
import jax.numpy as jnp
from jax import lax
from jax.experimental import pallas as pl
from jax.experimental.pallas import tpu as pltpu
from jax.experimental.pallas import tpu_sc as plsc

HIDDEN = 1024
LANES = 16
NTOK = 4 * 8192
NC, NS = 2, 16
NW = NC * NS
TPW = NTOK // NW
CH = 8
NCH = TPW // CH
RING = 4


def _emb_body(ids_hbm, pos_hbm, wtab_hbm, ptab_hbm, out_hbm,
              ids_v, pos_v, bufw, bufp, semw, semp, semo):
    c = lax.axis_index("c")
    s = lax.axis_index("s")
    wid = c * NS + s
    base = wid * TPW

    pltpu.sync_copy(ids_hbm.at[pl.ds(base, TPW)], ids_v)
    pltpu.sync_copy(pos_hbm.at[pl.ds(base, TPW)], pos_v)

    def fire_gathers(k, b):
        pltpu.async_copy(
            wtab_hbm.at[ids_v.at[pl.ds(k * CH, CH)]], bufw.at[b], semw[b])
        pltpu.async_copy(
            ptab_hbm.at[pos_v.at[pl.ds(k * CH, CH)]], bufp.at[b], semp[b])

    def wait_gathers(k, b):
        pltpu.make_async_copy(
            wtab_hbm.at[ids_v.at[pl.ds(k * CH, CH)]], bufw.at[b],
            semw[b]).wait()
        pltpu.make_async_copy(
            ptab_hbm.at[pos_v.at[pl.ds(k * CH, CH)]], bufp.at[b],
            semp[b]).wait()

    def fire_scatter(k, b):
        pltpu.async_copy(
            bufw.at[b], out_hbm.at[pl.ds(base + k * CH, CH)], semo[b])

    def wait_scatter(k, b):
        pltpu.make_async_copy(
            bufw.at[b], out_hbm.at[pl.ds(base + k * CH, CH)], semo[b]).wait()

    fire_gathers(0, 0)
    fire_gathers(1, 1)

    @pl.loop(0, NCH, step=RING)
    def _group(g):
        for b in range(RING):
            k = g + b
            wait_gathers(k, b)
            @pl.loop(0, CH)
            def _row(t):
                for i in range(HIDDEN // LANES):
                    sl = pl.ds(i * LANES, LANES)
                    bufw[b, t, sl] += bufp[b, t, sl]
            fire_scatter(k, b)
            nk = k + 2
            nb = (b + 2) % RING

            @pl.when(nk < NCH)
            def _prefetch():
                @pl.when(k >= 2)
                def _drain():
                    wait_scatter(k - 2, nb)
                fire_gathers(nk, nb)

    wait_scatter(NCH - 2, (NCH - 2) % RING)
    wait_scatter(NCH - 1, (NCH - 1) % RING)


@functools.partial(
    pl.kernel,
    out_type=jax.ShapeDtypeStruct((NTOK, HIDDEN), jnp.float32),
    mesh=plsc.VectorSubcoreMesh(core_axis_name="c", subcore_axis_name="s"),
    scratch_types=[
        pltpu.VMEM((TPW,), jnp.int32),
        pltpu.VMEM((TPW,), jnp.int32),
        pltpu.VMEM((RING, CH, HIDDEN), jnp.float32),
        pltpu.VMEM((RING, CH, HIDDEN), jnp.float32),
        [pltpu.SemaphoreType.DMA] * RING,
        [pltpu.SemaphoreType.DMA] * RING,
        [pltpu.SemaphoreType.DMA] * RING,
    ],
)
def _emb_call(ids_hbm, pos_hbm, wtab_hbm, ptab_hbm, out_hbm,
              ids_v, pos_v, bufw, bufp, semw, semp, semo):
    _emb_body(ids_hbm, pos_hbm, wtab_hbm, ptab_hbm, out_hbm,
              ids_v, pos_v, bufw, bufp, semw, semp, semo)


@jax.jit
def kernel(input_ids, position_ids, word_table, pos_table):
    bsh = input_ids.shape
    ids = input_ids.reshape(-1).astype(jnp.int32)
    pos = position_ids.reshape(-1).astype(jnp.int32)
    out = _emb_call(ids, pos, word_table, pos_table)
    return out.reshape(*bsh, HIDDEN)
